# baseline (speedup 1.0000x reference)
import jax
import jax.numpy as jnp
from jax.experimental import pallas as pl
from jax.experimental.pallas import tpu as pltpu

N = 1024
DIM = 128
NB = 8
BA = N // NB
PB = 512
W = 64
MAXCLS = 32
MARGIN = 0.2
CAP = 200
QBITS = 21
QMAX = (1 << QBITS) - 2
SCALE = float(1 << QBITS) / MARGIN
DEQUANT = MARGIN / float(1 << QBITS)
I32MAX = jnp.iinfo(jnp.int32).max
FQ = 16384.0
FQMAX = 16382.0
FSCALE = FQ / MARGIN
FDEQUANT = MARGIN / FQ
FBIG = 1e9
NPAD = N + 2 * BA


def _prep_kernel(lab_ref, lab8_ref, o_ref, ls_ref, ok_ref):
    labels = lab_ref[0, :]
    iota = jax.lax.broadcasted_iota(jnp.int32, (N,), 0)
    key = (labels * jnp.int32(N) + iota)[None, :]
    iota128 = jax.lax.broadcasted_iota(jnp.int32, (BA, 1), 0)

    def rank_body(c, acc):
        rank_acc, cs_acc = acc
        labch = lab8_ref[c, :].reshape(BA, 1)
        keych = labch * jnp.int32(N) + c * BA + iota128
        lt = (keych < key).astype(jnp.int32)
        eq = (labch == labels[None, :]).astype(jnp.int32)
        return (rank_acc + jnp.sum(lt, axis=0, keepdims=True),
                cs_acc + jnp.sum(eq, axis=0, keepdims=True))

    zero_row = jnp.zeros((1, N), jnp.int32)
    rank, csize = jax.lax.fori_loop(0, NB, rank_body, (zero_row, zero_row))

    r_row = iota[None, :]
    o_acc = zero_row
    ls_acc = zero_row
    for c in range(NB):
        rankch = rank[0, c * BA:(c + 1) * BA].reshape(BA, 1)
        labch = lab8_ref[c, :].reshape(BA, 1)
        sel = rankch == r_row
        o_acc = o_acc + jnp.sum(jnp.where(sel, c * BA + iota128, 0),
                                axis=0, keepdims=True)
        ls_acc = ls_acc + jnp.sum(jnp.where(sel, labch, 0),
                                  axis=0, keepdims=True)

    zpad = jnp.zeros((BA,), jnp.int32)
    o_ref[0, 0:BA] = zpad
    o_ref[0, BA:BA + N] = o_acc[0, :]
    o_ref[0, BA + N:NPAD] = zpad
    mpad = jnp.full((BA,), -1.0, jnp.float32)
    ls_ref[0, 0:BA] = mpad
    ls_ref[0, BA:BA + N] = ls_acc[0, :].astype(jnp.float32)
    ls_ref[0, BA + N:NPAD] = mpad

    ok = (jnp.max(csize) <= MAXCLS).astype(jnp.int32)
    ok_ref[0, :] = jnp.broadcast_to(ok, (128,))


def _halving_sum(d2):
    s = d2
    h = DIM
    while h > 1:
        h //= 2
        s = s[0:h] + s[h:2 * h]
    return s


def _mine_fast_kernel(embt_ref, emb_ref, lab_ref, o_ref, ls_ref,
                      t_ref, es_scr, ls_scr):
    i = pl.program_id(0)
    embt = embt_ref[:, :]
    labels = lab_ref[0, :]
    halo = 3 * BA // 2

    def gbody(rr, _):
        g = i * BA + BA - W // 2 + rr
        o_g = o_ref[0, g]
        es_scr[rr, :] = emb_ref[o_g, :]
        ls_scr[rr, :] = jnp.broadcast_to(ls_ref[0, g], (128,))
        return 0

    jax.lax.fori_loop(0, halo, gbody, 0)

    jpack = (jax.lax.broadcasted_iota(jnp.int32, (1, N), 1)
             .astype(jnp.float32) * jnp.float32(FQ))
    wiota = jax.lax.broadcasted_iota(jnp.int32, (W,), 0)

    def body(a, _):
        row_a = es_scr[W // 2 + a, :]
        la_f = ls_scr[W // 2 + a, 0]
        la_i = la_f.astype(jnp.int32)
        win = es_scr[pl.ds(a, W), :]
        wlab = ls_scr[pl.ds(a, W), 0]
        dwin = win - row_a[None, :]
        dp = jnp.sum(dwin * dwin, axis=1)
        pos_valid = (wlab == la_f) & (wiota != W // 2)

        diff = embt - row_a.reshape(DIM, 1)
        d_row = _halving_sum(diff * diff)
        d_neg = jnp.where(labels[None, :] != la_i, d_row, jnp.inf)

        dp_col = dp.reshape(W, 1)
        dpm_col = dp_col + jnp.float32(MARGIN)
        cond = (d_neg > dp_col) & (d_neg < dpm_col)
        q = jnp.minimum((dpm_col - d_neg) * jnp.float32(FSCALE),
                        jnp.float32(FQMAX))
        masked = jnp.where(cond, jpack + q, jnp.float32(FBIG))
        m = jnp.min(masked, axis=1)
        hit = m < jnp.float32(2.0e7)
        quant = m - jnp.floor(m * jnp.float32(1.0 / FQ)) * jnp.float32(FQ)
        term = quant * jnp.float32(FDEQUANT)
        valid = pos_valid & hit
        t_ref[a, :] = jnp.where(valid, term, jnp.float32(0.0))
        return 0

    jax.lax.fori_loop(0, BA, body, 0)


def _select_fast_kernel(t_ref, out_ref):
    t = t_ref[:, :]
    cnts = jnp.sum((t > 0.0).astype(jnp.int32), axis=1)
    sums = jnp.sum(t, axis=1)
    iota = jax.lax.broadcasted_iota(jnp.int32, (N,), 0)
    before = iota[None, :] < iota.reshape(N, 1)
    start = jnp.sum(jnp.where(before, cnts[None, :], 0), axis=1)
    total = jnp.sum(cnts)
    count = jnp.minimum(total, jnp.int32(CAP))

    full = (start + cnts) <= jnp.int32(CAP)
    loss_full = jnp.sum(jnp.where(full, sums, jnp.float32(0.0)))

    bmask = (start < CAP) & ((start + cnts) > CAP)
    has_b = jnp.any(bmask)
    r = jnp.int32(CAP) - jnp.sum(jnp.where(bmask, start, 0))
    row = jnp.sum(jnp.where(bmask.reshape(N, 1), t, jnp.float32(0.0)),
                  axis=0)
    v = row > 0.0
    wio = jax.lax.broadcasted_iota(jnp.int32, (W,), 0)
    le = wio[None, :] <= wio.reshape(W, 1)
    prefix = jnp.sum((le & v[None, :]).astype(jnp.int32), axis=1)
    include = v & (prefix <= r) & has_b
    loss_b = jnp.sum(jnp.where(include, row, jnp.float32(0.0)))

    loss = loss_full + loss_b
    outval = jnp.where(count > 0, loss / count.astype(jnp.float32),
                       jnp.float32(jnp.nan))
    out_ref[0, :] = jnp.broadcast_to(outval, (128,))


def _mine_dense_kernel(emb_blk_ref, emb_all_ref, lab_ref, t_ref):
    i = pl.program_id(0)
    emb_all = emb_all_ref[:, :]
    labels = lab_ref[0, :]
    jpacked = jax.lax.broadcasted_iota(jnp.int32, (PB, N), 1) << QBITS
    iota_n = jax.lax.broadcasted_iota(jnp.int32, (N,), 0)

    def body(a, _):
        e_a = emb_blk_ref[a, :]
        diff = emb_all - e_a[None, :]
        d_row = jnp.sum(diff * diff, axis=1)
        a_g = i * BA + a
        la = jnp.max(jnp.where(iota_n == a_g, labels, jnp.int32(-1)))
        neg = labels != la
        d_neg = jnp.where(neg, d_row, jnp.inf)[None, :]
        pos = (labels == la) & (iota_n != a_g)
        for c in range(N // PB):
            dp_col = d_row[c * PB:(c + 1) * PB].reshape(PB, 1)
            dpm_col = dp_col + jnp.float32(MARGIN)
            cond = (d_neg > dp_col) & (d_neg < dpm_col)
            quant = ((dpm_col - d_neg) * jnp.float32(SCALE)).astype(jnp.int32)
            quant = jnp.minimum(quant, QMAX)
            masked = jnp.where(cond, jpacked + quant, I32MAX)
            m = jnp.min(masked, axis=1)
            hit = m != I32MAX
            validc = pos[c * PB:(c + 1) * PB] & hit
            termc = ((m & ((1 << QBITS) - 1)) + 1).astype(jnp.float32) \
                * jnp.float32(DEQUANT)
            t_ref[a, pl.ds(c * PB, PB)] = jnp.where(validc, termc,
                                                    jnp.float32(0.0))
        return 0

    jax.lax.fori_loop(0, BA, body, 0)


def _select_dense_kernel(t_ref, lab_ref, out_ref):
    t = t_ref[:, :]
    cnts = jnp.sum((t > 0.0).astype(jnp.int32), axis=1)
    sums = jnp.sum(t, axis=1)
    la = lab_ref[0, :]
    iota = jax.lax.broadcasted_iota(jnp.int32, (N,), 0)
    k = la * jnp.int32(N) + iota
    before = k[None, :] < k.reshape(N, 1)
    start = jnp.sum(jnp.where(before, cnts[None, :], 0), axis=1)
    total = jnp.sum(cnts)
    count = jnp.minimum(total, jnp.int32(CAP))

    full = (start + cnts) <= jnp.int32(CAP)
    loss_full = jnp.sum(jnp.where(full, sums, jnp.float32(0.0)))

    bmask = (start < CAP) & ((start + cnts) > CAP)
    has_b = jnp.any(bmask)
    r = jnp.int32(CAP) - jnp.sum(jnp.where(bmask, start, 0))
    row = jnp.sum(jnp.where(bmask.reshape(N, 1), t, jnp.float32(0.0)), axis=0)
    v = row > 0.0
    le = iota[None, :] <= iota.reshape(N, 1)
    prefix = jnp.sum((le & v[None, :]).astype(jnp.int32), axis=1)
    include = v & (prefix <= r) & has_b
    loss_b = jnp.sum(jnp.where(include, row, jnp.float32(0.0)))

    loss = loss_full + loss_b
    outval = jnp.where(count > 0, loss / count.astype(jnp.float32),
                       jnp.float32(jnp.nan))
    out_ref[0, :] = jnp.broadcast_to(outval, (128,))


def _fast_path(embeddings, labels2, o, ls):
    halo = 3 * BA // 2
    t = pl.pallas_call(
        _mine_fast_kernel,
        grid=(NB,),
        in_specs=[
            pl.BlockSpec((DIM, N), lambda i: (0, 0)),
            pl.BlockSpec((N, DIM), lambda i: (0, 0)),
            pl.BlockSpec((1, N), lambda i: (0, 0)),
            pl.BlockSpec(memory_space=pltpu.SMEM),
            pl.BlockSpec(memory_space=pltpu.SMEM),
        ],
        out_specs=pl.BlockSpec((BA, W), lambda i: (i, 0)),
        out_shape=jax.ShapeDtypeStruct((N, W), jnp.float32),
        scratch_shapes=[
            pltpu.VMEM((halo, DIM), jnp.float32),
            pltpu.VMEM((halo, 128), jnp.float32),
        ],
        compiler_params=pltpu.CompilerParams(
            dimension_semantics=("parallel",)),
    )(embeddings.T, embeddings, labels2, o, ls)

    out = pl.pallas_call(
        _select_fast_kernel,
        out_shape=jax.ShapeDtypeStruct((1, 128), jnp.float32),
    )(t)
    return out[0, 0]


def _dense_path(embeddings, labels2):
    t = pl.pallas_call(
        _mine_dense_kernel,
        grid=(NB,),
        in_specs=[
            pl.BlockSpec((BA, DIM), lambda i: (i, 0)),
            pl.BlockSpec((N, DIM), lambda i: (0, 0)),
            pl.BlockSpec((1, N), lambda i: (0, 0)),
        ],
        out_specs=pl.BlockSpec((BA, N), lambda i: (i, 0)),
        out_shape=jax.ShapeDtypeStruct((N, N), jnp.float32),
    )(embeddings, embeddings, labels2)

    out = pl.pallas_call(
        _select_dense_kernel,
        out_shape=jax.ShapeDtypeStruct((1, 128), jnp.float32),
    )(t, labels2)
    return out[0, 0]


def kernel(embeddings, labels):
    labels2 = labels.astype(jnp.int32).reshape(1, N)
    o, ls, okv = pl.pallas_call(
        _prep_kernel,
        out_shape=[
            jax.ShapeDtypeStruct((1, NPAD), jnp.int32),
            jax.ShapeDtypeStruct((1, NPAD), jnp.float32),
            jax.ShapeDtypeStruct((1, 128), jnp.int32),
        ],
    )(labels2, labels2.reshape(NB, BA))
    return jax.lax.cond(
        okv[0, 0] > 0,
        lambda e, l, ov, lv: _fast_path(e, l, ov, lv),
        lambda e, l, ov, lv: _dense_path(e, l),
        embeddings, labels2, o, ls,
    )


# speedup vs baseline: 1.0051x; 1.0051x over previous
import jax
import jax.numpy as jnp
from jax.experimental import pallas as pl
from jax.experimental.pallas import tpu as pltpu

N = 1024
DIM = 128
NB = 8
BA = N // NB
PB = 512
W = 64
MAXCLS = 32
MARGIN = 0.2
CAP = 200
QBITS = 21
QMAX = (1 << QBITS) - 2
SCALE = float(1 << QBITS) / MARGIN
DEQUANT = MARGIN / float(1 << QBITS)
I32MAX = jnp.iinfo(jnp.int32).max
FQ = 16384.0
FQMAX = 16382.0
FSCALE = FQ / MARGIN
FDEQUANT = MARGIN / FQ
FBIG = 1e9
NPAD = N + 2 * BA


def _prep_kernel(lab_ref, lab8_ref, o_ref, ls_ref, ok_ref):
    labels = lab_ref[0, :]
    iota = jax.lax.broadcasted_iota(jnp.int32, (N,), 0)
    key = (labels * jnp.int32(N) + iota)[None, :]
    iota128 = jax.lax.broadcasted_iota(jnp.int32, (BA, 1), 0)

    def rank_body(c, acc):
        rank_acc, cs_acc = acc
        labch = lab8_ref[c, :].reshape(BA, 1)
        keych = labch * jnp.int32(N) + c * BA + iota128
        lt = (keych < key).astype(jnp.int32)
        eq = (labch == labels[None, :]).astype(jnp.int32)
        return (rank_acc + jnp.sum(lt, axis=0, keepdims=True),
                cs_acc + jnp.sum(eq, axis=0, keepdims=True))

    zero_row = jnp.zeros((1, N), jnp.int32)
    rank, csize = jax.lax.fori_loop(0, NB, rank_body, (zero_row, zero_row))

    r_row = iota[None, :]
    o_acc = zero_row
    ls_acc = zero_row
    for c in range(NB):
        rankch = rank[0, c * BA:(c + 1) * BA].reshape(BA, 1)
        labch = lab8_ref[c, :].reshape(BA, 1)
        sel = rankch == r_row
        o_acc = o_acc + jnp.sum(jnp.where(sel, c * BA + iota128, 0),
                                axis=0, keepdims=True)
        ls_acc = ls_acc + jnp.sum(jnp.where(sel, labch, 0),
                                  axis=0, keepdims=True)

    zpad = jnp.zeros((BA,), jnp.int32)
    o_ref[0, 0:BA] = zpad
    o_ref[0, BA:BA + N] = o_acc[0, :]
    o_ref[0, BA + N:NPAD] = zpad
    mpad = jnp.full((BA,), -1.0, jnp.float32)
    ls_ref[0, 0:BA] = mpad
    ls_ref[0, BA:BA + N] = ls_acc[0, :].astype(jnp.float32)
    ls_ref[0, BA + N:NPAD] = mpad

    ok = (jnp.max(csize) <= MAXCLS).astype(jnp.int32)
    ok_ref[0, :] = jnp.broadcast_to(ok, (128,))


def _halving_sum(d2):
    s = d2
    h = DIM
    while h > 1:
        h //= 2
        s = s[0:h] + s[h:2 * h]
    return s


def _mine_fast_kernel(embt_ref, emb_ref, lab_ref, o_ref, ls_ref,
                      t_ref, es_scr, ls_scr):
    i = pl.program_id(0)
    embt = embt_ref[:, :]
    labels = lab_ref[0, :]
    halo = 3 * BA // 2

    def gbody(rr, _):
        g = i * BA + BA - W // 2 + rr
        o_g = o_ref[0, g]
        es_scr[rr, :] = emb_ref[o_g, :]
        ls_scr[rr, :] = jnp.broadcast_to(ls_ref[0, g], (128,))
        return 0

    jax.lax.fori_loop(0, halo, gbody, 0)

    jpack = (jax.lax.broadcasted_iota(jnp.int32, (1, N), 1)
             .astype(jnp.float32) * jnp.float32(FQ))
    wiota = jax.lax.broadcasted_iota(jnp.int32, (W,), 0)

    def body(a, _):
        row_a = es_scr[W // 2 + a, :]
        la_f = ls_scr[W // 2 + a, 0]
        la_i = la_f.astype(jnp.int32)
        win = es_scr[pl.ds(a, W), :]
        wlab = ls_scr[pl.ds(a, W), 0]
        dwin = win - row_a[None, :]
        dp = jnp.sum(dwin * dwin, axis=1)
        pos_valid = (wlab == la_f) & (wiota != W // 2)

        diff = embt - row_a.reshape(DIM, 1)
        d_row = _halving_sum(diff * diff)
        d_neg = jnp.where(labels[None, :] != la_i, d_row, jnp.inf)

        dp_col = dp.reshape(W, 1)
        dpm_col = dp_col + jnp.float32(MARGIN)
        cond = (d_neg > dp_col) & (d_neg < dpm_col)
        q = jnp.minimum((dpm_col - d_neg) * jnp.float32(FSCALE),
                        jnp.float32(FQMAX))
        masked = jnp.where(cond, jpack + q, jnp.float32(FBIG))
        m = jnp.min(masked, axis=1)
        hit = m < jnp.float32(2.0e7)
        quant = m - jnp.floor(m * jnp.float32(1.0 / FQ)) * jnp.float32(FQ)
        term = quant * jnp.float32(FDEQUANT)
        valid = pos_valid & hit
        t_ref[a, :] = jnp.where(valid, term, jnp.float32(0.0))
        return 0

    jax.lax.fori_loop(0, BA, body, 0)


def _select_fast_kernel(t_ref, out_ref):
    t = t_ref[:, :]
    cnts = jnp.sum((t > 0.0).astype(jnp.int32), axis=1)
    sums = jnp.sum(t, axis=1)
    iota = jax.lax.broadcasted_iota(jnp.int32, (N,), 0)
    before = iota[None, :] < iota.reshape(N, 1)
    start = jnp.sum(jnp.where(before, cnts[None, :], 0), axis=1)
    total = jnp.sum(cnts)
    count = jnp.minimum(total, jnp.int32(CAP))

    full = (start + cnts) <= jnp.int32(CAP)
    loss_full = jnp.sum(jnp.where(full, sums, jnp.float32(0.0)))

    bmask = (start < CAP) & ((start + cnts) > CAP)
    has_b = jnp.any(bmask)
    r = jnp.int32(CAP) - jnp.sum(jnp.where(bmask, start, 0))
    row = jnp.sum(jnp.where(bmask.reshape(N, 1), t, jnp.float32(0.0)),
                  axis=0)
    v = row > 0.0
    wio = jax.lax.broadcasted_iota(jnp.int32, (W,), 0)
    le = wio[None, :] <= wio.reshape(W, 1)
    prefix = jnp.sum((le & v[None, :]).astype(jnp.int32), axis=1)
    include = v & (prefix <= r) & has_b
    loss_b = jnp.sum(jnp.where(include, row, jnp.float32(0.0)))

    loss = loss_full + loss_b
    outval = jnp.where(count > 0, loss / count.astype(jnp.float32),
                       jnp.float32(jnp.nan))
    out_ref[0, :] = jnp.broadcast_to(outval, (128,))


def _mine_dense_kernel(emb_blk_ref, emb_all_ref, lab_ref, t_ref):
    i = pl.program_id(0)
    emb_all = emb_all_ref[:, :]
    labels = lab_ref[0, :]
    jpacked = jax.lax.broadcasted_iota(jnp.int32, (PB, N), 1) << QBITS
    iota_n = jax.lax.broadcasted_iota(jnp.int32, (N,), 0)

    def body(a, _):
        e_a = emb_blk_ref[a, :]
        diff = emb_all - e_a[None, :]
        d_row = jnp.sum(diff * diff, axis=1)
        a_g = i * BA + a
        la = jnp.max(jnp.where(iota_n == a_g, labels, jnp.int32(-1)))
        neg = labels != la
        d_neg = jnp.where(neg, d_row, jnp.inf)[None, :]
        pos = (labels == la) & (iota_n != a_g)
        for c in range(N // PB):
            dp_col = d_row[c * PB:(c + 1) * PB].reshape(PB, 1)
            dpm_col = dp_col + jnp.float32(MARGIN)
            cond = (d_neg > dp_col) & (d_neg < dpm_col)
            quant = ((dpm_col - d_neg) * jnp.float32(SCALE)).astype(jnp.int32)
            quant = jnp.minimum(quant, QMAX)
            masked = jnp.where(cond, jpacked + quant, I32MAX)
            m = jnp.min(masked, axis=1)
            hit = m != I32MAX
            validc = pos[c * PB:(c + 1) * PB] & hit
            termc = ((m & ((1 << QBITS) - 1)) + 1).astype(jnp.float32) \
                * jnp.float32(DEQUANT)
            t_ref[a, pl.ds(c * PB, PB)] = jnp.where(validc, termc,
                                                    jnp.float32(0.0))
        return 0

    jax.lax.fori_loop(0, BA, body, 0)


def _select_dense_kernel(t_ref, lab_ref, out_ref):
    t = t_ref[:, :]
    cnts = jnp.sum((t > 0.0).astype(jnp.int32), axis=1)
    sums = jnp.sum(t, axis=1)
    la = lab_ref[0, :]
    iota = jax.lax.broadcasted_iota(jnp.int32, (N,), 0)
    k = la * jnp.int32(N) + iota
    before = k[None, :] < k.reshape(N, 1)
    start = jnp.sum(jnp.where(before, cnts[None, :], 0), axis=1)
    total = jnp.sum(cnts)
    count = jnp.minimum(total, jnp.int32(CAP))

    full = (start + cnts) <= jnp.int32(CAP)
    loss_full = jnp.sum(jnp.where(full, sums, jnp.float32(0.0)))

    bmask = (start < CAP) & ((start + cnts) > CAP)
    has_b = jnp.any(bmask)
    r = jnp.int32(CAP) - jnp.sum(jnp.where(bmask, start, 0))
    row = jnp.sum(jnp.where(bmask.reshape(N, 1), t, jnp.float32(0.0)), axis=0)
    v = row > 0.0
    le = iota[None, :] <= iota.reshape(N, 1)
    prefix = jnp.sum((le & v[None, :]).astype(jnp.int32), axis=1)
    include = v & (prefix <= r) & has_b
    loss_b = jnp.sum(jnp.where(include, row, jnp.float32(0.0)))

    loss = loss_full + loss_b
    outval = jnp.where(count > 0, loss / count.astype(jnp.float32),
                       jnp.float32(jnp.nan))
    out_ref[0, :] = jnp.broadcast_to(outval, (128,))


def _fast_path(embeddings, labels2, o, ls):
    halo = 3 * BA // 2
    t = pl.pallas_call(
        _mine_fast_kernel,
        grid=(NB,),
        in_specs=[
            pl.BlockSpec((DIM, N), lambda i: (0, 0)),
            pl.BlockSpec((N, DIM), lambda i: (0, 0)),
            pl.BlockSpec((1, N), lambda i: (0, 0)),
            pl.BlockSpec(memory_space=pltpu.SMEM),
            pl.BlockSpec(memory_space=pltpu.SMEM),
        ],
        out_specs=pl.BlockSpec((BA, W), lambda i: (i, 0)),
        out_shape=jax.ShapeDtypeStruct((N, W), jnp.float32),
        scratch_shapes=[
            pltpu.VMEM((halo, DIM), jnp.float32),
            pltpu.VMEM((halo, 128), jnp.float32),
        ],
    )(embeddings.T, embeddings, labels2, o, ls)

    out = pl.pallas_call(
        _select_fast_kernel,
        out_shape=jax.ShapeDtypeStruct((1, 128), jnp.float32),
    )(t)
    return out[0, 0]


def _dense_path(embeddings, labels2):
    t = pl.pallas_call(
        _mine_dense_kernel,
        grid=(NB,),
        in_specs=[
            pl.BlockSpec((BA, DIM), lambda i: (i, 0)),
            pl.BlockSpec((N, DIM), lambda i: (0, 0)),
            pl.BlockSpec((1, N), lambda i: (0, 0)),
        ],
        out_specs=pl.BlockSpec((BA, N), lambda i: (i, 0)),
        out_shape=jax.ShapeDtypeStruct((N, N), jnp.float32),
    )(embeddings, embeddings, labels2)

    out = pl.pallas_call(
        _select_dense_kernel,
        out_shape=jax.ShapeDtypeStruct((1, 128), jnp.float32),
    )(t, labels2)
    return out[0, 0]


def kernel(embeddings, labels):
    labels2 = labels.astype(jnp.int32).reshape(1, N)
    o, ls, okv = pl.pallas_call(
        _prep_kernel,
        out_shape=[
            jax.ShapeDtypeStruct((1, NPAD), jnp.int32),
            jax.ShapeDtypeStruct((1, NPAD), jnp.float32),
            jax.ShapeDtypeStruct((1, 128), jnp.int32),
        ],
    )(labels2, labels2.reshape(NB, BA))
    return jax.lax.cond(
        okv[0, 0] > 0,
        lambda e, l, ov, lv: _fast_path(e, l, ov, lv),
        lambda e, l, ov, lv: _dense_path(e, l),
        embeddings, labels2, o, ls,
    )
